# baseline (speedup 1.0000x reference)
import jax
import jax.numpy as jnp
from jax import lax
from jax.experimental import pallas as pl
from jax.experimental.pallas import tpu as pltpu
from jax.experimental.pallas import tpu_sc as plsc

_N = 4
_FEAT = 768
_PROJ = 512
_HID = 128
_B = 256
_BB = 16
_GRID = _B // _BB


def _dense_body(px_hbm, pt_ref, wext_ref, bext_ref, lng_ref, lnb_ref,
                wproj_ref, bproj_ref, wr1_ref, br1_ref, wr2_ref, br2_ref,
                stacked_ref, logits_ref, xbuf, sem):
    i = pl.program_id(0)
    slot = lax.rem(i, 2)

    @pl.when(i == 0)
    def _():
        pltpu.make_async_copy(px_hbm.at[pl.ds(0, _BB)], xbuf.at[0],
                              sem.at[0]).start()

    @pl.when(i + 1 < _GRID)
    def _():
        pltpu.make_async_copy(px_hbm.at[pl.ds((i + 1) * _BB, _BB)],
                              xbuf.at[1 - slot], sem.at[1 - slot]).start()

    pltpu.make_async_copy(px_hbm.at[pl.ds(i * _BB, _BB)], xbuf.at[slot],
                          sem.at[slot]).wait()

    hi = lax.Precision.HIGHEST
    x = xbuf[slot].reshape(_BB, 3, 14, 16, 224)
    xr = jnp.sum(x, axis=3)
    xr2 = xr.reshape(_BB * 42, 224)
    xp2 = jnp.dot(xr2, pt_ref[...], preferred_element_type=jnp.float32,
                  precision=hi)
    xp3 = xp2.reshape(_BB, 42, 14)
    xp = jnp.concatenate([xp3[:, g, :] for g in range(42)], axis=1)
    xp = xp * (1.0 / 256.0)

    ri_acc = None
    for n in range(_N):
        f = jnp.dot(xp, wext_ref[n], preferred_element_type=jnp.float32)
        f = f + bext_ref[n:n + 1, :]
        mu = jnp.mean(f, axis=-1, keepdims=True)
        var = jnp.mean(jnp.square(f - mu), axis=-1, keepdims=True)
        fn = (f - mu) * lax.rsqrt(var + 1e-5)
        fn = fn * lng_ref[n:n + 1, :] + lnb_ref[n:n + 1, :]
        p = jnp.dot(fn, wproj_ref[n], preferred_element_type=jnp.float32)
        p = p + bproj_ref[n:n + 1, :]
        stacked_ref[n] = p
        ri_acc = p if ri_acc is None else ri_acc + p

    ri = ri_acc * 0.25
    h = jnp.dot(ri, wr1_ref[...], preferred_element_type=jnp.float32)
    h = jnp.maximum(h + br1_ref[...], 0.0)
    logits = jnp.dot(h, wr2_ref[...], preferred_element_type=jnp.float32)
    logits_ref[...] = jnp.concatenate(
        [logits + br2_ref[...], jnp.zeros((_BB, 12), jnp.float32)], axis=1)


def _route_combine_body(stacked_hbm, logits_hbm, out_hbm, lbuf, sbuf,
                        obuf, sem):
    sid = lax.axis_index("s")
    cid = lax.axis_index("c")
    bb = sid * 16 + cid * 8

    copies = [
        pltpu.async_copy(
            stacked_hbm.at[n, pl.ds(bb, 8), :], sbuf.at[n], sem)
        for n in range(_N)
    ]
    pltpu.sync_copy(logits_hbm.at[pl.ds(bb, 8), :], lbuf)

    for c in copies:
        c.wait()

    for j in range(8):
        lv = lbuf[j]
        l = [jnp.full((16,), lv[n], jnp.float32) for n in range(_N)]
        m = jnp.maximum(jnp.maximum(l[0], l[1]), jnp.maximum(l[2], l[3]))
        e = [jnp.exp(v - m) for v in l]
        s = e[0] + e[1] + e[2] + e[3]
        p = [v / s for v in e]

        w = []
        for n in range(_N):
            cnt = jnp.zeros((16,), jnp.int32)
            for k in range(_N):
                if k == n:
                    continue
                beats = (l[k] >= l[n]) if k < n else (l[k] > l[n])
                cnt = cnt + jnp.where(beats, 1, 0)
            w.append(jnp.where(cnt < 2, p[n], 0.0))
        t = w[0] + w[1] + w[2] + w[3] + 1e-8
        w = [v / t for v in w]

        for c in range(_PROJ // 16):
            acc = None
            for n in range(_N):
                v = w[n] * sbuf[n, j, pl.ds(c * 16, 16)]
                acc = v if acc is None else acc + v
            obuf[j, pl.ds(c * 16, 16)] = acc

    pltpu.sync_copy(obuf, out_hbm.at[pl.ds(bb, 8), :])


def kernel(pixel_values, W_ext, b_ext, ln_g, ln_b, W_proj, b_proj,
           W_r1, b_r1, W_r2, b_r2):
    pt = jnp.repeat(jnp.eye(14, dtype=jnp.float32), 16, axis=0)
    br1 = b_r1.reshape(1, _HID)
    br2 = b_r2.reshape(1, _N)

    stacked, logits = pl.pallas_call(
        _dense_body,
        grid=(_GRID,),
        in_specs=[
            pl.BlockSpec(memory_space=pl.ANY),
            pl.BlockSpec((224, 14), lambda i: (0, 0)),
            pl.BlockSpec((_N, 588, _FEAT), lambda i: (0, 0, 0)),
            pl.BlockSpec((_N, _FEAT), lambda i: (0, 0)),
            pl.BlockSpec((_N, _FEAT), lambda i: (0, 0)),
            pl.BlockSpec((_N, _FEAT), lambda i: (0, 0)),
            pl.BlockSpec((_N, _FEAT, _PROJ), lambda i: (0, 0, 0)),
            pl.BlockSpec((_N, _PROJ), lambda i: (0, 0)),
            pl.BlockSpec((_PROJ, _HID), lambda i: (0, 0)),
            pl.BlockSpec((1, _HID), lambda i: (0, 0)),
            pl.BlockSpec((_HID, _N), lambda i: (0, 0)),
            pl.BlockSpec((1, _N), lambda i: (0, 0)),
        ],
        out_specs=[
            pl.BlockSpec((_N, _BB, _PROJ), lambda i: (0, i, 0)),
            pl.BlockSpec((_BB, 16), lambda i: (i, 0)),
        ],
        out_shape=[
            jax.ShapeDtypeStruct((_N, _B, _PROJ), jnp.float32),
            jax.ShapeDtypeStruct((_B, 16), jnp.float32),
        ],
        scratch_shapes=[
            pltpu.VMEM((2, _BB, 3, 224, 224), jnp.float32),
            pltpu.SemaphoreType.DMA((2,)),
        ],
        compiler_params=pltpu.CompilerParams(
            dimension_semantics=("arbitrary",)),
    )(pixel_values, pt, W_ext, b_ext, ln_g, ln_b, W_proj, b_proj,
      W_r1, br1, W_r2, br2)

    route_combine = pl.kernel(
        _route_combine_body,
        mesh=plsc.VectorSubcoreMesh(core_axis_name="c", subcore_axis_name="s"),
        out_type=jax.ShapeDtypeStruct((_B, _PROJ), jnp.float32),
        scratch_types=[
            pltpu.VMEM((8, 16), jnp.float32),
            pltpu.VMEM((_N, 8, _PROJ), jnp.float32),
            pltpu.VMEM((8, _PROJ), jnp.float32),
            pltpu.SemaphoreType.DMA,
        ],
    )
    return route_combine(stacked, logits)


# speedup vs baseline: 1.1300x; 1.1300x over previous
import jax
import jax.numpy as jnp
from jax import lax
from jax.experimental import pallas as pl
from jax.experimental.pallas import tpu as pltpu
from jax.experimental.pallas import tpu_sc as plsc

_N = 4
_FEAT = 768
_PROJ = 512
_HID = 128
_B = 256
_BB = 16
_GRID = _B // _BB


def _dense_body(x_ref, pt_ref, wext_ref, bext_ref, lng_ref, lnb_ref,
                wproj_ref, bproj_ref, wr1_ref, br1_ref, wr2_ref, br2_ref,
                stacked_ref, logits_ref):
    hi = lax.Precision.HIGHEST
    x = x_ref[...]
    xr = jnp.sum(x, axis=3)
    xr2 = xr.reshape(_BB * 42, 224)
    xp2 = jnp.dot(xr2, pt_ref[...], preferred_element_type=jnp.float32,
                  precision=hi)
    xp3 = xp2.reshape(_BB, 42, 14)
    xp = jnp.concatenate([xp3[:, g, :] for g in range(42)], axis=1)
    xp = xp * (1.0 / 256.0)

    ri_acc = None
    for n in range(_N):
        f = jnp.dot(xp, wext_ref[n], preferred_element_type=jnp.float32)
        f = f + bext_ref[n:n + 1, :]
        mu = jnp.mean(f, axis=-1, keepdims=True)
        var = jnp.mean(jnp.square(f - mu), axis=-1, keepdims=True)
        fn = (f - mu) * lax.rsqrt(var + 1e-5)
        fn = fn * lng_ref[n:n + 1, :] + lnb_ref[n:n + 1, :]
        p = jnp.dot(fn, wproj_ref[n], preferred_element_type=jnp.float32)
        p = p + bproj_ref[n:n + 1, :]
        stacked_ref[n] = p
        ri_acc = p if ri_acc is None else ri_acc + p

    ri = ri_acc * 0.25
    h = jnp.dot(ri, wr1_ref[...], preferred_element_type=jnp.float32)
    h = jnp.maximum(h + br1_ref[...], 0.0)
    logits = jnp.dot(h, wr2_ref[...], preferred_element_type=jnp.float32)
    logits_ref[...] = logits + br2_ref[...]


_DG = _PROJ // 16


def _route_combine_body(stackedT_hbm, logitsT_hbm, outT_hbm, lbuf, buf, obuf,
                        sem):
    sid = lax.axis_index("s")
    cid = lax.axis_index("c")
    d0 = sid * _DG
    b0 = cid * 128

    copies = [
        pltpu.async_copy(
            stackedT_hbm.at[n, pl.ds(d0, _DG), pl.ds(b0, 128)],
            buf.at[n], sem)
        for n in range(_N)
    ]
    pltpu.sync_copy(logitsT_hbm.at[:, pl.ds(b0, 128)], lbuf)
    for c in copies:
        c.wait()

    for k in range(8):
        l = [lbuf[n, pl.ds(k * 16, 16)] for n in range(_N)]

        m = jnp.maximum(jnp.maximum(l[0], l[1]), jnp.maximum(l[2], l[3]))
        e = [jnp.exp(v - m) for v in l]
        s = e[0] + e[1] + e[2] + e[3]
        p = [v / s for v in e]

        w = []
        for n in range(_N):
            cnt = jnp.zeros((16,), jnp.int32)
            for j in range(_N):
                if j == n:
                    continue
                beats = (l[j] >= l[n]) if j < n else (l[j] > l[n])
                cnt = cnt + jnp.where(beats, 1, 0)
            w.append(jnp.where(cnt < 2, p[n], 0.0))
        t = w[0] + w[1] + w[2] + w[3] + 1e-8
        w = [v / t for v in w]

        for d in range(_DG):
            acc = None
            for n in range(_N):
                v = buf[n, d, pl.ds(k * 16, 16)]
                acc = w[n] * v if acc is None else acc + w[n] * v
            obuf[d, pl.ds(k * 16, 16)] = acc

    pltpu.sync_copy(obuf, outT_hbm.at[pl.ds(d0, _DG), pl.ds(b0, 128)])


def kernel(pixel_values, W_ext, b_ext, ln_g, ln_b, W_proj, b_proj,
           W_r1, b_r1, W_r2, b_r2):
    x5 = pixel_values.reshape(_B, 3, 14, 16, 224)
    pt = jnp.repeat(jnp.eye(14, dtype=jnp.float32), 16, axis=0)
    br1 = b_r1.reshape(1, _HID)
    br2 = b_r2.reshape(1, _N)

    stacked, logits = pl.pallas_call(
        _dense_body,
        grid=(_GRID,),
        in_specs=[
            pl.BlockSpec((_BB, 3, 14, 16, 224), lambda i: (i, 0, 0, 0, 0)),
            pl.BlockSpec((224, 14), lambda i: (0, 0)),
            pl.BlockSpec((_N, 588, _FEAT), lambda i: (0, 0, 0)),
            pl.BlockSpec((_N, _FEAT), lambda i: (0, 0)),
            pl.BlockSpec((_N, _FEAT), lambda i: (0, 0)),
            pl.BlockSpec((_N, _FEAT), lambda i: (0, 0)),
            pl.BlockSpec((_N, _FEAT, _PROJ), lambda i: (0, 0, 0)),
            pl.BlockSpec((_N, _PROJ), lambda i: (0, 0)),
            pl.BlockSpec((_PROJ, _HID), lambda i: (0, 0)),
            pl.BlockSpec((1, _HID), lambda i: (0, 0)),
            pl.BlockSpec((_HID, _N), lambda i: (0, 0)),
            pl.BlockSpec((1, _N), lambda i: (0, 0)),
        ],
        out_specs=[
            pl.BlockSpec((_N, _BB, _PROJ), lambda i: (0, i, 0)),
            pl.BlockSpec((_BB, _N), lambda i: (i, 0)),
        ],
        out_shape=[
            jax.ShapeDtypeStruct((_N, _B, _PROJ), jnp.float32),
            jax.ShapeDtypeStruct((_B, _N), jnp.float32),
        ],
    )(x5, pt, W_ext, b_ext, ln_g, ln_b, W_proj, b_proj, W_r1, br1, W_r2, br2)

    route_combine = pl.kernel(
        _route_combine_body,
        mesh=plsc.VectorSubcoreMesh(core_axis_name="c", subcore_axis_name="s"),
        out_type=jax.ShapeDtypeStruct((_PROJ, _B), jnp.float32),
        scratch_types=[
            pltpu.VMEM((_N, 128), jnp.float32),
            pltpu.VMEM((_N, _DG, 128), jnp.float32),
            pltpu.VMEM((_DG, 128), jnp.float32),
            pltpu.SemaphoreType.DMA,
        ],
    )
    stackedT = jnp.transpose(stacked, (0, 2, 1))
    logitsT = logits.T
    fusedT = route_combine(stackedT, logitsT)
    return fusedT.T
